# baseline (speedup 1.0000x reference)
import functools

import jax
import jax.numpy as jnp
from jax import lax
from jax.experimental import pallas as pl
from jax.experimental.pallas import tpu as pltpu
from jax.experimental.pallas import tpu_sc as plsc

_NC = 2
_NS = 16
_NW = _NC * _NS
_GROUP = 128
_NBUF = 2


@functools.lru_cache(maxsize=None)
def _make_lookup(n, v, d):
    assert n % (_NW * _GROUP) == 0
    b_per_w = n // _NW
    groups_per_w = b_per_w // _GROUP

    max_rows = (170 * 1024) // (4 * d)
    g_chunk = 1
    for g in range(1, min(groups_per_w, 12) + 1):
        if (groups_per_w % g == 0
                and (groups_per_w // g) % _NBUF == 0
                and g * _GROUP <= max_rows):
            g_chunk = g
    C = g_chunk * _GROUP
    n_chunks = b_per_w // C

    mesh = plsc.VectorSubcoreMesh(core_axis_name="c", subcore_axis_name="s")

    @functools.partial(
        pl.kernel,
        mesh=mesh,
        out_type=jax.ShapeDtypeStruct((n, d), jnp.float32),
        scratch_types=[
            pltpu.VMEM((_NBUF, g_chunk, _GROUP), jnp.int32),
            pltpu.VMEM((_NBUF, C, d), jnp.float32),
            pltpu.SemaphoreType.DMA,
            pltpu.SemaphoreType.DMA,
            pltpu.SemaphoreType.DMA,
        ],
        compiler_params=pltpu.CompilerParams(use_tc_tiling_on_sc=False),
    )
    def emb_kernel(idx_hbm, tab_hbm, out_hbm, idx_v, rows_v, gsem,
                   wsem0, wsem1):
        wid = lax.axis_index("s") * _NC + lax.axis_index("c")
        base = wid * b_per_w
        chunk0 = wid * n_chunks
        wsems = (wsem0, wsem1)

        def do_chunk(ci, b, wait_write):
            if wait_write:
                pltpu.make_async_copy(
                    rows_v.at[b], out_hbm.at[pl.ds(base, C)], wsems[b]
                ).wait()
            pltpu.sync_copy(idx_hbm.at[chunk0 + ci], idx_v.at[b])
            copies = [
                pltpu.async_copy(tab_hbm.at[idx_v.at[b].at[g]],
                                 rows_v.at[b].at[pl.ds(g * _GROUP, _GROUP)],
                                 gsem)
                for g in range(g_chunk)
            ]
            for cpy in copies:
                cpy.wait()
            pltpu.async_copy(rows_v.at[b],
                             out_hbm.at[pl.ds(base + ci * C, C)],
                             wsems[b])

        for b in range(_NBUF):
            do_chunk(b, b, wait_write=False)

        def body(c0, carry):
            for b in range(_NBUF):
                do_chunk(c0 + b, b, wait_write=True)
            return carry

        if n_chunks > _NBUF:
            lax.fori_loop(0, (n_chunks - _NBUF) // _NBUF,
                          lambda i, c: body(_NBUF + i * _NBUF, c), 0)

        for b in range(_NBUF):
            pltpu.make_async_copy(
                rows_v.at[b], out_hbm.at[pl.ds(base, C)], wsems[b]
            ).wait()

    return emb_kernel, g_chunk


def kernel(token_id, table):
    b, l = token_id.shape
    v, d = table.shape
    n = b * l
    lookup, g_chunk = _make_lookup(n, v, d)
    idx3d = token_id.reshape(-1, g_chunk, _GROUP).astype(jnp.int32)
    out = lookup(idx3d, table)
    return out.reshape(b, l, d)


# speedup vs baseline: 1.0012x; 1.0012x over previous
import functools

import jax
import jax.numpy as jnp
from jax import lax
from jax.experimental import pallas as pl
from jax.experimental.pallas import tpu as pltpu
from jax.experimental.pallas import tpu_sc as plsc

_NC = 2
_NS = 16
_NW = _NC * _NS
_NBUF = 2


@functools.lru_cache(maxsize=None)
def _make_lookup(b, l, v, d):
    assert b % _NW == 0
    rows_pw = b // _NW

    segs = []
    off = 0
    while l - off > 0:
        seg = min(128, l - off)
        segs.append((off, seg))
        off += seg

    k = 1
    for cand in range(1, rows_pw + 1):
        if (rows_pw % cand == 0
            and (rows_pw // cand) % _NBUF == 0
            and cand * len(segs) <= 16
                and cand * l * (4 + 4 * d) * _NBUF <= 420 * 1024):
            k = cand
    n_chunks = rows_pw // k

    mesh = plsc.VectorSubcoreMesh(core_axis_name="c", subcore_axis_name="s")

    @functools.partial(
        pl.kernel,
        mesh=mesh,
        out_type=jax.ShapeDtypeStruct((b, l, d), jnp.float32),
        scratch_types=[
            pltpu.VMEM((_NBUF, k, l), jnp.int32),
            pltpu.VMEM((_NBUF, k, l, d), jnp.float32),
            pltpu.SemaphoreType.DMA,
            pltpu.SemaphoreType.DMA,
            pltpu.SemaphoreType.DMA,
        ],
        compiler_params=pltpu.CompilerParams(use_tc_tiling_on_sc=False),
    )
    def emb_kernel(idx_hbm, tab_hbm, out_hbm, idx_v, rows_v, gsem,
                   wsem0, wsem1):
        wid = lax.axis_index("s") * _NC + lax.axis_index("c")
        row0 = wid * rows_pw
        wsems = (wsem0, wsem1)

        def do_chunk(ci, slot, wait_write):
            if wait_write:
                pltpu.make_async_copy(
                    rows_v.at[slot], out_hbm.at[pl.ds(row0, k)], wsems[slot]
                ).wait()
            pltpu.sync_copy(idx_hbm.at[pl.ds(row0 + ci * k, k)],
                            idx_v.at[slot])
            copies = [
                pltpu.async_copy(
                    tab_hbm.at[idx_v.at[slot].at[r].at[pl.ds(o, s)]],
                    rows_v.at[slot].at[r].at[pl.ds(o, s)],
                    gsem)
                for r in range(k)
                for (o, s) in segs
            ]
            for cpy in copies:
                cpy.wait()
            pltpu.async_copy(rows_v.at[slot],
                             out_hbm.at[pl.ds(row0 + ci * k, k)],
                             wsems[slot])

        for slot in range(_NBUF):
            do_chunk(slot, slot, wait_write=False)

        def body(c0, carry):
            for slot in range(_NBUF):
                do_chunk(c0 + slot, slot, wait_write=True)
            return carry

        if n_chunks > _NBUF:
            lax.fori_loop(0, (n_chunks - _NBUF) // _NBUF,
                          lambda i, c: body(_NBUF + i * _NBUF, c), 0)

        for slot in range(_NBUF):
            pltpu.make_async_copy(
                rows_v.at[slot], out_hbm.at[pl.ds(row0, k)], wsems[slot]
            ).wait()

    return emb_kernel


def kernel(token_id, table):
    b, l = token_id.shape
    v, d = table.shape
    if token_id.dtype != jnp.int32:
        token_id = token_id.astype(jnp.int32)
    return _make_lookup(b, l, v, d)(token_id, table)
